# Initial kernel scaffold; baseline (speedup 1.0000x reference)
import jax
import jax.numpy as jnp
from jax.experimental import pallas as pl


def kernel(hidden_state, W_action, b_action, D):
    raise NotImplementedError("write your pallas kernel here")


# speedup vs baseline: 160.4884x; 160.4884x over previous
import jax
import jax.numpy as jnp
from jax.experimental import pallas as pl

B, S, H, DEPTH = 1, 512, 1024, 32
TS = 64


def _body(hs_ref, w_ref, b_ref, out_ref):
    hs = hs_ref[...]
    acc = jnp.dot(hs, w_ref[...], preferred_element_type=jnp.float32,
                  precision=jax.lax.Precision.HIGHEST)
    acc = acc + b_ref[...]
    cols = jax.lax.broadcasted_iota(jnp.int32, acc.shape, 1)
    is_logit = cols < 3
    lm = jnp.where(is_logit, acc, -1e30)
    mx = jnp.max(lm, axis=1, keepdims=True)
    e = jnp.where(is_logit, jnp.exp(lm - mx), 0.0)
    push = e[:, 0:1] / jnp.sum(e, axis=1, keepdims=True)
    d = acc[:, 3:4]
    c = push * (1.0 / (1.0 + jnp.exp(-d)))
    out_ref[...] = jnp.zeros(out_ref.shape, jnp.float32)
    out_ref[:, 0, :] = jnp.broadcast_to(c, (out_ref.shape[0], out_ref.shape[2]))


def kernel(hidden_state, W_action, b_action, D):
    hs = hidden_state.reshape(S, H)
    wd = jnp.zeros((H, 8), jnp.float32).at[:, :3].set(W_action.T).at[:, 3].set(D[0])
    bp = jnp.zeros((1, 8), jnp.float32).at[0, :3].set(b_action)

    out = pl.pallas_call(
        _body,
        grid=(S // TS,),
        in_specs=[
            pl.BlockSpec((TS, H), lambda i: (i, 0)),
            pl.BlockSpec((H, 8), lambda i: (0, 0)),
            pl.BlockSpec((1, 8), lambda i: (0, 0)),
        ],
        out_specs=pl.BlockSpec((TS, DEPTH, H), lambda i: (i, 0, 0)),
        out_shape=jax.ShapeDtypeStruct((S, DEPTH, H), jnp.float32),
    )(hs, wd, bp)
    return out.reshape(B, S, DEPTH, H)
